# baseline (speedup 1.0000x reference)
import functools

import jax
import jax.numpy as jnp
from jax import lax
from jax.experimental import pallas as pl
from jax.experimental.pallas import tpu as pltpu
from jax.experimental.pallas import tpu_sc as plsc

NC = 2
NS = 16
NW = NC * NS

NE = 1000000
DIM = 16
NPLANE = 64
BT = 8
PL_W = NPLANE // NW


SPAN = 62
NBLK2 = 2 * 126
TPR = 7813
NT = 2 * TPR
DPITCH = TPR * 1024


def _detile_body(ent3, tails, out3, buf, tbuf, wsem):
    wid = lax.axis_index("s") * NC + lax.axis_index("c")
    def per_blk(j, _):
        b = wid + j * NW
        @pl.when(b < NBLK2)
        def _():
            dt = lax.div(b, 126)
            blk = lax.rem(b, 126)
            pltpu.sync_copy(ent3.at[dt, :, pl.ds(blk * (SPAN * 128),
                                                 SPAN * 128)], buf)
            t0 = dt * TPR + blk * SPAN
            def per_t(c, _):
                pltpu.async_copy(buf.at[:, pl.ds(c * 128, 128)],
                                 out3.at[t0 + c], wsem)
                return 0
            lax.fori_loop(0, SPAN, per_t, 0)
            pltpu.make_async_copy(ent3.at[0, :, pl.ds(0, SPAN * 128)],
                                  buf, wsem).wait()
        return 0
    lax.fori_loop(0, 8, per_blk, 0)

    @pl.when(wid < 2)
    def _():
        pltpu.sync_copy(tails.at[wid], tbuf)
        pltpu.sync_copy(tbuf, out3.at[wid * TPR + TPR - 1])


def _sc_body(ent_hbm, rel_hbm, h_hbm, r_hbm, t_hbm, v_hbm,
             hs_out, rs_out, ts_out, vs_out,
             relv, idxv, ridxv, sidx, ebuf, vidxv, vbuf, rbufs,
             gsem, vsem, wsem):
    wid = lax.axis_index("s") * NC + lax.axis_index("c")

    pltpu.sync_copy(rel_hbm, relv)

    def entity_issue(src_hbm, p):
        pltpu.sync_copy(src_hbm.at[p], idxv)
        def per_bt_x(bt, _):
            for g in range(8):
                v = idxv[bt, pl.ds(g * 16, 16)]
                idxv[bt, pl.ds(g * 16, 16)] = v + (v >> 7) * 896
            return 0
        lax.fori_loop(0, BT, per_bt_x, 0)
        def per_bt(bt, _):
            for dt in range(2):
                def per_d(ds_, _, dt=dt):
                    dpart = dt * DPITCH + ds_ * 128
                    for g in range(8):
                        sidx[bt, dt * 8 + ds_, pl.ds(g * 16, 16)] = (
                            idxv[bt, pl.ds(g * 16, 16)] + dpart)
                    pltpu.async_copy(ent_hbm.at[sidx.at[bt, dt * 8 + ds_]],
                                     ebuf.at[dt, bt, ds_], gsem)
                    return 0
                lax.fori_loop(0, 8, per_d, 0)
            return 0
        lax.fori_loop(0, BT, per_bt, 0)

    def entity_finish(dst_hbm, p):
        pltpu.make_async_copy(dst_hbm.at[p], ebuf, gsem).wait()
        pltpu.sync_copy(ebuf, dst_hbm.at[p])

    def vs_work():
        pltpu.sync_copy(v_hbm.at[wid], vidxv)
        for g in range(8):
            v = vidxv[0, pl.ds(g * 16, 16)]
            vidxv[0, pl.ds(g * 16, 16)] = v + (v >> 7) * 896
        for dt in range(2):
            def per_d(ds_, _, dt=dt):
                dpart = dt * DPITCH + ds_ * 128
                for g in range(8):
                    sidx[0, dt * 8 + ds_, pl.ds(g * 16, 16)] = (
                        vidxv[0, pl.ds(g * 16, 16)] + dpart)
                pltpu.async_copy(ent_hbm.at[sidx.at[0, dt * 8 + ds_]],
                                 vbuf.at[dt, ds_], vsem)
                return 0
            lax.fori_loop(0, 8, per_d, 0)
        for dt in range(2):
            pltpu.make_async_copy(vs_out.at[dt, 0], vbuf.at[dt], vsem).wait()
        for dt in range(2):
            pltpu.sync_copy(vbuf.at[dt], vs_out.at[dt, wid])

    rbuf = rbufs.at[0]

    def drain_rs_writes():
        def per_i(i, _):
            for jt in range(2):
                pltpu.make_async_copy(rs_out.at[0, 0, 0, 0],
                                      rbuf.at[0, pl.ds(jt * 8, 8)],
                                      wsem).wait()
            return 0
        lax.fori_loop(0, 16, per_i, 0)

    def rs_half(p, bt0):
        def per_bt(q, _):
            bt = bt0 + q
            def per_group(g, _):
                rv = ridxv[bt, g] * 256
                def per_i(i, _):
                    base = rv + i * 16
                    for j in range(16):
                        vals = plsc.load_gather(relv, [base + j])
                        rbuf[i, j, pl.ds(g * 16, 16)] = vals
                    return 0
                lax.fori_loop(0, DIM, per_i, 0)
                return 0
            lax.fori_loop(0, 8, per_group, 0)
            def per_write(i, _):
                for jt in range(2):
                    pltpu.async_copy(rbuf.at[i, pl.ds(jt * 8, 8)],
                                     rs_out.at[p, i, jt, bt], wsem)
                return 0
            lax.fori_loop(0, 16, per_write, 0)
            drain_rs_writes()
            return 0
        lax.fori_loop(0, 4, per_bt, 0)

    for k in range(PL_W):
        p = wid * PL_W + k
        entity_issue(h_hbm, p)
        entity_finish(hs_out, p)
        entity_issue(t_hbm, p)
        entity_finish(ts_out, p)
        pltpu.sync_copy(r_hbm.at[p], ridxv)
        rs_half(p, 0)
        rs_half(p, 4)

    @pl.when(wid < BT)
    def _():
        vs_work()


@jax.jit
def _run(ent3, tails, rel_t, h_lin, r_lin, t_lin, v_lin):
    mesh = plsc.VectorSubcoreMesh(core_axis_name="c", subcore_axis_name="s")
    detile = functools.partial(
        pl.kernel,
        mesh=mesh,
        out_type=[jax.ShapeDtypeStruct((NT, 8, 128), jnp.float32)],
        scratch_types=[pltpu.VMEM((8, SPAN * 128), jnp.float32),
                       pltpu.VMEM((8, 128), jnp.float32),
                       pltpu.SemaphoreType.DMA],
    )(_detile_body)
    (out3,) = detile(ent3, tails)
    ent_flat = out3.reshape(NT * 1024)
    f = functools.partial(
        pl.kernel,
        mesh=mesh,
        compiler_params=pltpu.CompilerParams(needs_layout_passes=False),
        out_type=[
            jax.ShapeDtypeStruct((NPLANE, 2, BT, 8, 128), jnp.float32),
            jax.ShapeDtypeStruct((NPLANE, 16, 2, BT, 8, 128), jnp.float32),
            jax.ShapeDtypeStruct((NPLANE, 2, BT, 8, 128), jnp.float32),
            jax.ShapeDtypeStruct((2, BT, 8, 128), jnp.float32),
        ],
        scratch_types=[
            pltpu.VMEM((256 * 32,), jnp.float32),
            pltpu.VMEM((BT, 128), jnp.int32),
            pltpu.VMEM((BT, 8, 16), jnp.int32),
            pltpu.VMEM((BT, DIM, 128), jnp.int32),
            pltpu.VMEM((2, BT, 8, 128), jnp.float32),
            pltpu.VMEM((1, 128), jnp.int32),
            pltpu.VMEM((2, 8, 128), jnp.float32),
            pltpu.VMEM((1, 16, 16, 128), jnp.float32),
            pltpu.SemaphoreType.DMA,
            pltpu.SemaphoreType.DMA,
            pltpu.SemaphoreType.DMA,
        ],
    )(_sc_body)
    return f(ent_flat, rel_t, h_lin, r_lin, t_lin, v_lin)


def kernel(h_i, R_i, t_i, v_i, entity_emb, relation_emb):
    ent3 = jnp.transpose(entity_emb).reshape(2, 8, NE)
    tails = jnp.pad(
        jnp.transpose(entity_emb[7812 * 128:]).reshape(2, 8, 64),
        ((0, 0), (0, 0), (0, 64)))
    rel_t = relation_emb.reshape(32 * 256)
    h_lin = jnp.transpose(h_i, (1, 2, 0)).reshape(NPLANE, BT, 128)
    r_lin = jnp.transpose(R_i, (1, 2, 0)).reshape(NPLANE, BT, 8, 16)
    t_lin = jnp.transpose(t_i, (1, 2, 0)).reshape(NPLANE, BT, 128)
    v_lin = v_i.reshape(BT, 1, 128)
    hz, rz, tz, vz = _run(ent3, tails, rel_t, h_lin, r_lin, t_lin, v_lin)
    hs = (hz.reshape(2, 32, 2, BT, 8, 128)
            .transpose(3, 5, 0, 1, 2, 4).reshape(1024, 2, 32, DIM))
    rs = (rz.reshape(2, 32, 16, 2, BT, 8, 128)
            .transpose(4, 6, 0, 1, 2, 3, 5).reshape(1024, 2, 32, DIM, DIM))
    ts = (tz.reshape(2, 32, 2, BT, 8, 128)
            .transpose(3, 5, 0, 1, 2, 4).reshape(1024, 2, 32, DIM))
    vs = vz.transpose(1, 3, 0, 2).reshape(1024, DIM)
    return (hs, rs, ts, vs)


# speedup vs baseline: 1.8087x; 1.8087x over previous
import functools

import jax
import jax.numpy as jnp
from jax import lax
from jax.experimental import pallas as pl
from jax.experimental.pallas import tpu as pltpu
from jax.experimental.pallas import tpu_sc as plsc

NC = 2
NS = 16
NW = NC * NS

NE = 1000000
DIM = 16
NPLANE = 64
BT = 8
PL_W = NPLANE // NW


SPAN = 62
NBLK2 = 2 * 126
TPR = 7813
NT = 2 * TPR
DPITCH = TPR * 1024


def _detile_body(ent3, tails, out3, buf, tbuf, wsem):
    wid = lax.axis_index("s") * NC + lax.axis_index("c")
    def per_blk(j, _):
        b = wid + j * NW
        @pl.when(b < NBLK2)
        def _():
            dt = lax.div(b, 126)
            blk = lax.rem(b, 126)
            pltpu.sync_copy(ent3.at[dt, :, pl.ds(blk * (SPAN * 128),
                                                 SPAN * 128)], buf)
            t0 = dt * TPR + blk * SPAN
            def per_t(c, _):
                pltpu.async_copy(buf.at[:, pl.ds(c * 128, 128)],
                                 out3.at[t0 + c], wsem)
                return 0
            lax.fori_loop(0, SPAN, per_t, 0)
            pltpu.make_async_copy(ent3.at[0, :, pl.ds(0, SPAN * 128)],
                                  buf, wsem).wait()
        return 0
    lax.fori_loop(0, 8, per_blk, 0)

    @pl.when(wid < 2)
    def _():
        pltpu.sync_copy(tails.at[wid], tbuf)
        pltpu.sync_copy(tbuf, out3.at[wid * TPR + TPR - 1])


def _sc_body(ent_hbm, rel_hbm, h_hbm, r_hbm, t_hbm, v_hbm,
             hs_out, rs_out, ts_out, vs_out,
             relv, idxv, ridxv, sidx, ebuf, vidxv, vbuf, rbufs,
             gsem, vsem, wsem):
    wid = lax.axis_index("s") * NC + lax.axis_index("c")

    pltpu.sync_copy(rel_hbm, relv)

    def entity_issue(src_hbm, p):
        pltpu.sync_copy(src_hbm.at[p], idxv)
        def per_bt_x(bt, _):
            for g in range(8):
                v = idxv[bt, pl.ds(g * 16, 16)]
                idxv[bt, pl.ds(g * 16, 16)] = v + (v >> 7) * 896
            return 0
        lax.fori_loop(0, BT, per_bt_x, 0)
        def per_bt(bt, _):
            for dt in range(2):
                def per_d(ds_, _, dt=dt):
                    dpart = dt * DPITCH + ds_ * 128
                    for g in range(8):
                        sidx[bt, dt * 8 + ds_, pl.ds(g * 16, 16)] = (
                            idxv[bt, pl.ds(g * 16, 16)] + dpart)
                    pltpu.async_copy(ent_hbm.at[sidx.at[bt, dt * 8 + ds_]],
                                     ebuf.at[dt, bt, ds_], gsem)
                    return 0
                lax.fori_loop(0, 8, per_d, 0)
            return 0
        lax.fori_loop(0, BT, per_bt, 0)

    def entity_finish(dst_hbm, p):
        pltpu.make_async_copy(dst_hbm.at[p], ebuf, gsem).wait()
        pltpu.sync_copy(ebuf, dst_hbm.at[p])

    def vs_work():
        pltpu.sync_copy(v_hbm.at[wid], vidxv)
        for g in range(8):
            v = vidxv[0, pl.ds(g * 16, 16)]
            vidxv[0, pl.ds(g * 16, 16)] = v + (v >> 7) * 896
        for dt in range(2):
            def per_d(ds_, _, dt=dt):
                dpart = dt * DPITCH + ds_ * 128
                for g in range(8):
                    sidx[0, dt * 8 + ds_, pl.ds(g * 16, 16)] = (
                        vidxv[0, pl.ds(g * 16, 16)] + dpart)
                pltpu.async_copy(ent_hbm.at[sidx.at[0, dt * 8 + ds_]],
                                 vbuf.at[dt, ds_], vsem)
                return 0
            lax.fori_loop(0, 8, per_d, 0)
        for dt in range(2):
            pltpu.make_async_copy(vs_out.at[dt, 0], vbuf.at[dt], vsem).wait()
        for dt in range(2):
            pltpu.sync_copy(vbuf.at[dt], vs_out.at[dt, wid])

    rbuf = rbufs.at[0]

    def drain_rs_writes():
        def per_i(i, _):
            for jt in range(2):
                pltpu.make_async_copy(rs_out.at[0, 0, 0, 0],
                                      rbuf.at[0, pl.ds(jt * 8, 8)],
                                      wsem).wait()
            return 0
        lax.fori_loop(0, 16, per_i, 0)

    def rs_half(p, bt0):
        def per_bt(q, _):
            bt = bt0 + q
            def per_group(g, _):
                rvec = ridxv[bt, g]
                def per_i(i, _):
                    row = i * 16
                    for j in range(16):
                        vals = plsc.load_gather(
                            relv,
                            [jnp.full((16,), row + j, jnp.int32), rvec])
                        rbuf[i, j, pl.ds(g * 16, 16)] = vals
                    return 0
                lax.fori_loop(0, DIM, per_i, 0)
                return 0
            lax.fori_loop(0, 8, per_group, 0)
            def per_write(i, _):
                for jt in range(2):
                    pltpu.async_copy(rbuf.at[i, pl.ds(jt * 8, 8)],
                                     rs_out.at[p, i, jt, bt], wsem)
                return 0
            lax.fori_loop(0, 16, per_write, 0)
            drain_rs_writes()
            return 0
        lax.fori_loop(0, 4, per_bt, 0)

    for k in range(PL_W):
        p = wid * PL_W + k
        entity_issue(h_hbm, p)
        entity_finish(hs_out, p)
        entity_issue(t_hbm, p)
        entity_finish(ts_out, p)
        pltpu.sync_copy(r_hbm.at[p], ridxv)
        rs_half(p, 0)
        rs_half(p, 4)

    @pl.when(wid < BT)
    def _():
        vs_work()


@jax.jit
def _run(ent3, tails, rel_t, h_lin, r_lin, t_lin, v_lin):
    mesh = plsc.VectorSubcoreMesh(core_axis_name="c", subcore_axis_name="s")
    detile = functools.partial(
        pl.kernel,
        mesh=mesh,
        out_type=[jax.ShapeDtypeStruct((NT, 8, 128), jnp.float32)],
        scratch_types=[pltpu.VMEM((8, SPAN * 128), jnp.float32),
                       pltpu.VMEM((8, 128), jnp.float32),
                       pltpu.SemaphoreType.DMA],
    )(_detile_body)
    (out3,) = detile(ent3, tails)
    ent_flat = out3.reshape(NT * 1024)
    f = functools.partial(
        pl.kernel,
        mesh=mesh,
        compiler_params=pltpu.CompilerParams(needs_layout_passes=False),
        out_type=[
            jax.ShapeDtypeStruct((NPLANE, 2, BT, 8, 128), jnp.float32),
            jax.ShapeDtypeStruct((NPLANE, 16, 2, BT, 8, 128), jnp.float32),
            jax.ShapeDtypeStruct((NPLANE, 2, BT, 8, 128), jnp.float32),
            jax.ShapeDtypeStruct((2, BT, 8, 128), jnp.float32),
        ],
        scratch_types=[
            pltpu.VMEM((256, 32), jnp.float32),
            pltpu.VMEM((BT, 128), jnp.int32),
            pltpu.VMEM((BT, 8, 16), jnp.int32),
            pltpu.VMEM((BT, DIM, 128), jnp.int32),
            pltpu.VMEM((2, BT, 8, 128), jnp.float32),
            pltpu.VMEM((1, 128), jnp.int32),
            pltpu.VMEM((2, 8, 128), jnp.float32),
            pltpu.VMEM((1, 16, 16, 128), jnp.float32),
            pltpu.SemaphoreType.DMA,
            pltpu.SemaphoreType.DMA,
            pltpu.SemaphoreType.DMA,
        ],
    )(_sc_body)
    return f(ent_flat, rel_t, h_lin, r_lin, t_lin, v_lin)


def kernel(h_i, R_i, t_i, v_i, entity_emb, relation_emb):
    ent3 = jnp.transpose(entity_emb).reshape(2, 8, NE)
    tails = jnp.pad(
        jnp.transpose(entity_emb[7812 * 128:]).reshape(2, 8, 64),
        ((0, 0), (0, 0), (0, 64)))
    rel_t = jnp.transpose(relation_emb, (1, 2, 0)).reshape(256, 32)
    h_lin = jnp.transpose(h_i, (1, 2, 0)).reshape(NPLANE, BT, 128)
    r_lin = jnp.transpose(R_i, (1, 2, 0)).reshape(NPLANE, BT, 8, 16)
    t_lin = jnp.transpose(t_i, (1, 2, 0)).reshape(NPLANE, BT, 128)
    v_lin = v_i.reshape(BT, 1, 128)
    hz, rz, tz, vz = _run(ent3, tails, rel_t, h_lin, r_lin, t_lin, v_lin)
    hs = (hz.reshape(2, 32, 2, BT, 8, 128)
            .transpose(3, 5, 0, 1, 2, 4).reshape(1024, 2, 32, DIM))
    rs = (rz.reshape(2, 32, 16, 2, BT, 8, 128)
            .transpose(4, 6, 0, 1, 2, 3, 5).reshape(1024, 2, 32, DIM, DIM))
    ts = (tz.reshape(2, 32, 2, BT, 8, 128)
            .transpose(3, 5, 0, 1, 2, 4).reshape(1024, 2, 32, DIM))
    vs = vz.transpose(1, 3, 0, 2).reshape(1024, DIM)
    return (hs, rs, ts, vs)


# speedup vs baseline: 1.9146x; 1.0586x over previous
import functools

import jax
import jax.numpy as jnp
from jax import lax
from jax.experimental import pallas as pl
from jax.experimental.pallas import tpu as pltpu
from jax.experimental.pallas import tpu_sc as plsc

NC = 2
NS = 16
NW = NC * NS

NE = 1000000
DIM = 16
NPLANE = 64
BT = 8
PL_W = NPLANE // NW


SPAN = 62
NBLK2 = 2 * 126
TPR = 7813
NT = 2 * TPR
DPITCH = TPR * 1024


def _detile_body(ent3, tails, out3, buf, tbuf, wsem):
    wid = lax.axis_index("s") * NC + lax.axis_index("c")
    def per_blk(j, _):
        b = wid + j * NW
        @pl.when(b < NBLK2)
        def _():
            dt = lax.div(b, 126)
            blk = lax.rem(b, 126)
            pltpu.sync_copy(ent3.at[dt, :, pl.ds(blk * (SPAN * 128),
                                                 SPAN * 128)], buf)
            t0 = dt * TPR + blk * SPAN
            def per_t(c, _):
                pltpu.async_copy(buf.at[:, pl.ds(c * 128, 128)],
                                 out3.at[t0 + c], wsem)
                return 0
            lax.fori_loop(0, SPAN, per_t, 0)
            pltpu.make_async_copy(ent3.at[0, :, pl.ds(0, SPAN * 128)],
                                  buf, wsem).wait()
        return 0
    lax.fori_loop(0, 8, per_blk, 0)

    @pl.when(wid < 2)
    def _():
        pltpu.sync_copy(tails.at[wid], tbuf)
        pltpu.sync_copy(tbuf, out3.at[wid * TPR + TPR - 1])


def _sc_body(ent_hbm, rel_hbm, h_hbm, r_hbm, t_hbm, v_hbm,
             hs_out, rs_out, ts_out, vs_out,
             relv, idxv, ridxv, sidx, ebuf, vidxv, vbuf, rbufs,
             gsem, vsem, wsem):
    wid = lax.axis_index("s") * NC + lax.axis_index("c")

    pltpu.sync_copy(rel_hbm, relv)

    def entity_issue(src_hbm, p):
        pltpu.sync_copy(src_hbm.at[p], idxv)
        def per_bt_x(bt, _):
            for g in range(8):
                v = idxv[bt, pl.ds(g * 16, 16)]
                idxv[bt, pl.ds(g * 16, 16)] = v + (v >> 7) * 896
            return 0
        lax.fori_loop(0, BT, per_bt_x, 0)
        def per_bt(bt, _):
            for dt in range(2):
                def per_d(ds_, _, dt=dt):
                    dpart = dt * DPITCH + ds_ * 128
                    for g in range(8):
                        sidx[bt, dt * 8 + ds_, pl.ds(g * 16, 16)] = (
                            idxv[bt, pl.ds(g * 16, 16)] + dpart)
                    pltpu.async_copy(ent_hbm.at[sidx.at[bt, dt * 8 + ds_]],
                                     ebuf.at[dt, bt, ds_], gsem)
                    return 0
                lax.fori_loop(0, 8, per_d, 0)
            return 0
        lax.fori_loop(0, BT, per_bt, 0)

    def entity_finish(dst_hbm, p):
        pltpu.make_async_copy(dst_hbm.at[p], ebuf, gsem).wait()
        pltpu.sync_copy(ebuf, dst_hbm.at[p])

    def vs_work():
        pltpu.sync_copy(v_hbm.at[wid], vidxv)
        for g in range(8):
            v = vidxv[0, pl.ds(g * 16, 16)]
            vidxv[0, pl.ds(g * 16, 16)] = v + (v >> 7) * 896
        for dt in range(2):
            def per_d(ds_, _, dt=dt):
                dpart = dt * DPITCH + ds_ * 128
                for g in range(8):
                    sidx[0, dt * 8 + ds_, pl.ds(g * 16, 16)] = (
                        vidxv[0, pl.ds(g * 16, 16)] + dpart)
                pltpu.async_copy(ent_hbm.at[sidx.at[0, dt * 8 + ds_]],
                                 vbuf.at[dt, ds_], vsem)
                return 0
            lax.fori_loop(0, 8, per_d, 0)
        for dt in range(2):
            pltpu.make_async_copy(vs_out.at[dt, 0], vbuf.at[dt], vsem).wait()
        for dt in range(2):
            pltpu.sync_copy(vbuf.at[dt], vs_out.at[dt, wid])

    rbuf = rbufs.at[0]

    def drain_rs_writes():
        def per_i(i, _):
            for jt in range(2):
                pltpu.make_async_copy(rs_out.at[0, 0, 0, 0],
                                      rbuf.at[0, pl.ds(jt * 8, 8)],
                                      wsem).wait()
            return 0
        lax.fori_loop(0, 16, per_i, 0)

    def rs_half(p, bt0):
        def per_bt(q, _):
            bt = bt0 + q
            def per_group(g, _):
                rvec = ridxv[bt, g]
                def per_i(i, _):
                    row = i * 16
                    for j in range(16):
                        vals = plsc.load_gather(
                            relv,
                            [jnp.full((16,), row + j, jnp.int32), rvec])
                        rbuf[i, j, pl.ds(g * 16, 16)] = vals
                    return 0
                lax.fori_loop(0, DIM, per_i, 0)
                return 0
            lax.fori_loop(0, 8, per_group, 0)
            def per_write(i, _):
                for jt in range(2):
                    pltpu.async_copy(rbuf.at[i, pl.ds(jt * 8, 8)],
                                     rs_out.at[p, i, jt, bt], wsem)
                return 0
            lax.fori_loop(0, 16, per_write, 0)
            drain_rs_writes()
            return 0
        lax.fori_loop(0, 4, per_bt, 0)

    for k in range(PL_W):
        p = wid * PL_W + k
        entity_issue(h_hbm, p)
        pltpu.sync_copy(r_hbm.at[p], ridxv)
        rs_half(p, 0)
        entity_finish(hs_out, p)
        entity_issue(t_hbm, p)
        rs_half(p, 4)
        entity_finish(ts_out, p)

    @pl.when(wid < BT)
    def _():
        vs_work()


@jax.jit
def _run(ent3, tails, rel_t, h_lin, r_lin, t_lin, v_lin):
    mesh = plsc.VectorSubcoreMesh(core_axis_name="c", subcore_axis_name="s")
    detile = functools.partial(
        pl.kernel,
        mesh=mesh,
        out_type=[jax.ShapeDtypeStruct((NT, 8, 128), jnp.float32)],
        scratch_types=[pltpu.VMEM((8, SPAN * 128), jnp.float32),
                       pltpu.VMEM((8, 128), jnp.float32),
                       pltpu.SemaphoreType.DMA],
    )(_detile_body)
    (out3,) = detile(ent3, tails)
    ent_flat = out3.reshape(NT * 1024)
    f = functools.partial(
        pl.kernel,
        mesh=mesh,
        compiler_params=pltpu.CompilerParams(needs_layout_passes=False),
        out_type=[
            jax.ShapeDtypeStruct((NPLANE, 2, BT, 8, 128), jnp.float32),
            jax.ShapeDtypeStruct((NPLANE, 16, 2, BT, 8, 128), jnp.float32),
            jax.ShapeDtypeStruct((NPLANE, 2, BT, 8, 128), jnp.float32),
            jax.ShapeDtypeStruct((2, BT, 8, 128), jnp.float32),
        ],
        scratch_types=[
            pltpu.VMEM((256, 32), jnp.float32),
            pltpu.VMEM((BT, 128), jnp.int32),
            pltpu.VMEM((BT, 8, 16), jnp.int32),
            pltpu.VMEM((BT, DIM, 128), jnp.int32),
            pltpu.VMEM((2, BT, 8, 128), jnp.float32),
            pltpu.VMEM((1, 128), jnp.int32),
            pltpu.VMEM((2, 8, 128), jnp.float32),
            pltpu.VMEM((1, 16, 16, 128), jnp.float32),
            pltpu.SemaphoreType.DMA,
            pltpu.SemaphoreType.DMA,
            pltpu.SemaphoreType.DMA,
        ],
    )(_sc_body)
    return f(ent_flat, rel_t, h_lin, r_lin, t_lin, v_lin)


def kernel(h_i, R_i, t_i, v_i, entity_emb, relation_emb):
    ent3 = jnp.transpose(entity_emb).reshape(2, 8, NE)
    tails = jnp.pad(
        jnp.transpose(entity_emb[7812 * 128:]).reshape(2, 8, 64),
        ((0, 0), (0, 0), (0, 64)))
    rel_t = jnp.transpose(relation_emb, (1, 2, 0)).reshape(256, 32)
    h_lin = jnp.transpose(h_i, (1, 2, 0)).reshape(NPLANE, BT, 128)
    r_lin = jnp.transpose(R_i, (1, 2, 0)).reshape(NPLANE, BT, 8, 16)
    t_lin = jnp.transpose(t_i, (1, 2, 0)).reshape(NPLANE, BT, 128)
    v_lin = v_i.reshape(BT, 1, 128)
    hz, rz, tz, vz = _run(ent3, tails, rel_t, h_lin, r_lin, t_lin, v_lin)
    hs = (hz.reshape(2, 32, 2, BT, 8, 128)
            .transpose(3, 5, 0, 1, 2, 4).reshape(1024, 2, 32, DIM))
    rs = (rz.reshape(2, 32, 16, 2, BT, 8, 128)
            .transpose(4, 6, 0, 1, 2, 3, 5).reshape(1024, 2, 32, DIM, DIM))
    ts = (tz.reshape(2, 32, 2, BT, 8, 128)
            .transpose(3, 5, 0, 1, 2, 4).reshape(1024, 2, 32, DIM))
    vs = vz.transpose(1, 3, 0, 2).reshape(1024, DIM)
    return (hs, rs, ts, vs)


# speedup vs baseline: 1.9150x; 1.0002x over previous
import functools

import jax
import jax.numpy as jnp
from jax import lax
from jax.experimental import pallas as pl
from jax.experimental.pallas import tpu as pltpu
from jax.experimental.pallas import tpu_sc as plsc

NC = 2
NS = 16
NW = NC * NS

NE = 1000000
DIM = 16
NPLANE = 64
BT = 8
PL_W = NPLANE // NW


SPAN = 62
NBLK2 = 2 * 126
TPR = 7813
NT = 2 * TPR
DPITCH = TPR * 1024


def _detile_body(ent3, tails, out3, buf, tbuf, wsem):
    wid = lax.axis_index("s") * NC + lax.axis_index("c")
    def per_blk(j, _):
        b = wid + j * NW
        @pl.when(b < NBLK2)
        def _():
            dt = lax.div(b, 126)
            blk = lax.rem(b, 126)
            pltpu.sync_copy(ent3.at[dt, :, pl.ds(blk * (SPAN * 128),
                                                 SPAN * 128)], buf)
            t0 = dt * TPR + blk * SPAN
            def per_t(c, _):
                pltpu.async_copy(buf.at[:, pl.ds(c * 128, 128)],
                                 out3.at[t0 + c], wsem)
                return 0
            lax.fori_loop(0, SPAN, per_t, 0)
            pltpu.make_async_copy(ent3.at[0, :, pl.ds(0, SPAN * 128)],
                                  buf, wsem).wait()
        return 0
    lax.fori_loop(0, 8, per_blk, 0)

    @pl.when(wid < 2)
    def _():
        pltpu.sync_copy(tails.at[wid], tbuf)
        pltpu.sync_copy(tbuf, out3.at[wid * TPR + TPR - 1])


def _sc_body(ent_hbm, rel_hbm, h_hbm, r_hbm, t_hbm, v_hbm,
             hs_out, rs_out, ts_out, vs_out,
             relv, idxv, ridxv, sidx, ebuf, vidxv, vbuf, rbufs,
             gsem, vsem, wsem):
    wid = lax.axis_index("s") * NC + lax.axis_index("c")

    pltpu.sync_copy(rel_hbm, relv)

    def entity_issue(src_hbm, p):
        pltpu.sync_copy(src_hbm.at[p], idxv)
        def per_bt_x(bt, _):
            for g in range(8):
                v = idxv[bt, pl.ds(g * 16, 16)]
                idxv[bt, pl.ds(g * 16, 16)] = v + (v >> 7) * 896
            return 0
        lax.fori_loop(0, BT, per_bt_x, 0)
        def per_bt(bt, _):
            for dt in range(2):
                def per_d(ds_, _, dt=dt):
                    dpart = dt * DPITCH + ds_ * 128
                    for g in range(8):
                        sidx[bt, dt * 8 + ds_, pl.ds(g * 16, 16)] = (
                            idxv[bt, pl.ds(g * 16, 16)] + dpart)
                    pltpu.async_copy(ent_hbm.at[sidx.at[bt, dt * 8 + ds_]],
                                     ebuf.at[dt, bt, ds_], gsem)
                    return 0
                lax.fori_loop(0, 8, per_d, 0)
            return 0
        lax.fori_loop(0, BT, per_bt, 0)

    def entity_finish(dst_hbm, p):
        pltpu.make_async_copy(dst_hbm.at[p], ebuf, gsem).wait()
        pltpu.sync_copy(ebuf, dst_hbm.at[p])

    def vs_work():
        pltpu.sync_copy(v_hbm.at[wid], vidxv)
        for g in range(8):
            v = vidxv[0, pl.ds(g * 16, 16)]
            vidxv[0, pl.ds(g * 16, 16)] = v + (v >> 7) * 896
        for dt in range(2):
            def per_d(ds_, _, dt=dt):
                dpart = dt * DPITCH + ds_ * 128
                for g in range(8):
                    sidx[0, dt * 8 + ds_, pl.ds(g * 16, 16)] = (
                        vidxv[0, pl.ds(g * 16, 16)] + dpart)
                pltpu.async_copy(ent_hbm.at[sidx.at[0, dt * 8 + ds_]],
                                 vbuf.at[dt, ds_], vsem)
                return 0
            lax.fori_loop(0, 8, per_d, 0)
        for dt in range(2):
            pltpu.make_async_copy(vs_out.at[dt, 0], vbuf.at[dt], vsem).wait()
        for dt in range(2):
            pltpu.sync_copy(vbuf.at[dt], vs_out.at[dt, wid])

    rbuf = rbufs.at[0]

    def drain_rs_writes():
        def per_i(i, _):
            for jt in range(2):
                pltpu.make_async_copy(rs_out.at[0, 0, 0, 0],
                                      rbuf.at[0, pl.ds(jt * 8, 8)],
                                      wsem).wait()
            return 0
        lax.fori_loop(0, 16, per_i, 0)

    def rs_half(p, bt0):
        def per_bt(q, _):
            bt = bt0 + q
            def per_group(g, _):
                rvec = ridxv[bt, g]
                def per_i(i, _):
                    row = i * 16
                    for j in range(16):
                        vals = plsc.load_gather(
                            relv,
                            [jnp.full((16,), row + j, jnp.int32), rvec])
                        rbuf[i, j, pl.ds(g * 16, 16)] = vals
                    return 0
                lax.fori_loop(0, DIM, per_i, 0)
                return 0
            lax.fori_loop(0, 8, per_group, 0)
            def per_write(i, _):
                for jt in range(2):
                    pltpu.async_copy(rbuf.at[i, pl.ds(jt * 8, 8)],
                                     rs_out.at[p, i, jt, bt], wsem)
                return 0
            lax.fori_loop(0, 16, per_write, 0)
            drain_rs_writes()
            return 0
        lax.fori_loop(0, 4, per_bt, 0)

    for k in range(PL_W):
        p = wid * PL_W + k
        with jax.named_scope("ph_issue_h"):
            entity_issue(h_hbm, p)
            pltpu.sync_copy(r_hbm.at[p], ridxv)
        with jax.named_scope("ph_rs_a"):
            rs_half(p, 0)
        with jax.named_scope("ph_fin_h"):
            entity_finish(hs_out, p)
        with jax.named_scope("ph_issue_t"):
            entity_issue(t_hbm, p)
        with jax.named_scope("ph_rs_b"):
            rs_half(p, 4)
        with jax.named_scope("ph_fin_t"):
            entity_finish(ts_out, p)

    @pl.when(wid < BT)
    def _():
        vs_work()


@jax.jit
def _run(ent3, tails, rel_t, h_lin, r_lin, t_lin, v_lin):
    mesh = plsc.VectorSubcoreMesh(core_axis_name="c", subcore_axis_name="s")
    detile = functools.partial(
        pl.kernel,
        mesh=mesh,
        out_type=[jax.ShapeDtypeStruct((NT, 8, 128), jnp.float32)],
        scratch_types=[pltpu.VMEM((8, SPAN * 128), jnp.float32),
                       pltpu.VMEM((8, 128), jnp.float32),
                       pltpu.SemaphoreType.DMA],
    )(_detile_body)
    (out3,) = detile(ent3, tails)
    ent_flat = out3.reshape(NT * 1024)
    f = functools.partial(
        pl.kernel,
        mesh=mesh,
        compiler_params=pltpu.CompilerParams(needs_layout_passes=False),
        out_type=[
            jax.ShapeDtypeStruct((NPLANE, 2, BT, 8, 128), jnp.float32),
            jax.ShapeDtypeStruct((NPLANE, 16, 2, BT, 8, 128), jnp.float32),
            jax.ShapeDtypeStruct((NPLANE, 2, BT, 8, 128), jnp.float32),
            jax.ShapeDtypeStruct((2, BT, 8, 128), jnp.float32),
        ],
        scratch_types=[
            pltpu.VMEM((256, 32), jnp.float32),
            pltpu.VMEM((BT, 128), jnp.int32),
            pltpu.VMEM((BT, 8, 16), jnp.int32),
            pltpu.VMEM((BT, DIM, 128), jnp.int32),
            pltpu.VMEM((2, BT, 8, 128), jnp.float32),
            pltpu.VMEM((1, 128), jnp.int32),
            pltpu.VMEM((2, 8, 128), jnp.float32),
            pltpu.VMEM((1, 16, 16, 128), jnp.float32),
            pltpu.SemaphoreType.DMA,
            pltpu.SemaphoreType.DMA,
            pltpu.SemaphoreType.DMA,
        ],
    )(_sc_body)
    return f(ent_flat, rel_t, h_lin, r_lin, t_lin, v_lin)


def kernel(h_i, R_i, t_i, v_i, entity_emb, relation_emb):
    ent3 = jnp.transpose(entity_emb).reshape(2, 8, NE)
    tails = jnp.pad(
        jnp.transpose(entity_emb[7812 * 128:]).reshape(2, 8, 64),
        ((0, 0), (0, 0), (0, 64)))
    rel_t = jnp.transpose(relation_emb, (1, 2, 0)).reshape(256, 32)
    h_lin = jnp.transpose(h_i, (1, 2, 0)).reshape(NPLANE, BT, 128)
    r_lin = jnp.transpose(R_i, (1, 2, 0)).reshape(NPLANE, BT, 8, 16)
    t_lin = jnp.transpose(t_i, (1, 2, 0)).reshape(NPLANE, BT, 128)
    v_lin = v_i.reshape(BT, 1, 128)
    hz, rz, tz, vz = _run(ent3, tails, rel_t, h_lin, r_lin, t_lin, v_lin)
    hs = (hz.reshape(2, 32, 2, BT, 8, 128)
            .transpose(3, 5, 0, 1, 2, 4).reshape(1024, 2, 32, DIM))
    rs = (rz.reshape(2, 32, 16, 2, BT, 8, 128)
            .transpose(4, 6, 0, 1, 2, 3, 5).reshape(1024, 2, 32, DIM, DIM))
    ts = (tz.reshape(2, 32, 2, BT, 8, 128)
            .transpose(3, 5, 0, 1, 2, 4).reshape(1024, 2, 32, DIM))
    vs = vz.transpose(1, 3, 0, 2).reshape(1024, DIM)
    return (hs, rs, ts, vs)
